# baseline (speedup 1.0000x reference)
import functools

import jax
import jax.numpy as jnp
from jax import lax
from jax.experimental import pallas as pl
from jax.experimental.pallas import tpu as pltpu
from jax.experimental.pallas import tpu_sc as plsc

_N = 10000
_E = 320000
_D = 128
_K = 64

_NC = 2
_NS = 16
_NW = _NC * _NS
_C = 128
_NCH = 80
_EPW = _NCH * _C
_EPAD = _NW * _EPW
_NPAD = 10240
_RPT = _NPAD // _NS

_mesh = plsc.VectorSubcoreMesh(core_axis_name="c", subcore_axis_name="s")


@functools.partial(
    pl.kernel,
    out_type=jax.ShapeDtypeStruct((_NC * _NPAD,), jnp.float32),
    mesh=_mesh,
    scratch_types=[
        pltpu.VMEM((_NCH, _C), jnp.int32),
        pltpu.VMEM((_C,), jnp.float32),
        pltpu.VMEM((_RPT,), jnp.float32),
        pltpu.VMEM_SHARED((_NPAD,), jnp.float32),
    ],
)
def _deg_kernel(dst_hbm, out_hbm, idxd_v, ones_v, zbuf_v, part_sh):
    cid = lax.axis_index("c")
    sid = lax.axis_index("s")
    wid = sid * _NC + cid

    def fo(k, _):
        ones_v[pl.ds(k * 16, 16)] = jnp.ones((16,), jnp.float32)
        return 0
    lax.fori_loop(0, _C // 16, fo, 0)

    def fz(k, _):
        zbuf_v[pl.ds(k * 16, 16)] = jnp.zeros((16,), jnp.float32)
        return 0
    lax.fori_loop(0, _RPT // 16, fz, 0)

    pltpu.sync_copy(zbuf_v, part_sh.at[pl.ds(sid * _RPT, _RPT)])
    pltpu.sync_copy(dst_hbm.at[wid], idxd_v)
    plsc.subcore_barrier()

    def chunk(j, _):
        pltpu.sync_copy(ones_v, part_sh.at[idxd_v.at[j]], add=True)
        return 0
    lax.fori_loop(0, _NCH, chunk, 0)
    plsc.subcore_barrier()

    pltpu.sync_copy(part_sh.at[pl.ds(sid * _RPT, _RPT)],
                    out_hbm.at[pl.ds(cid * _NPAD + sid * _RPT, _RPT)])


@functools.partial(
    pl.kernel,
    out_type=jax.ShapeDtypeStruct((_NC, _NPAD, _D), jnp.float32),
    mesh=_mesh,
    scratch_types=[
        pltpu.VMEM((_C,), jnp.int32),
        pltpu.VMEM((_C,), jnp.int32),
        pltpu.VMEM((_C,), jnp.int32),
        pltpu.VMEM((_C,), jnp.int32),
        pltpu.VMEM((_C, _D), jnp.float32),
        pltpu.VMEM((_C, _D), jnp.float32),
        pltpu.VMEM_SHARED((_NPAD, _D), jnp.float32),
        pltpu.SemaphoreType.DMA,
        pltpu.SemaphoreType.DMA,
        pltpu.SemaphoreType.DMA,
        pltpu.SemaphoreType.DMA,
        pltpu.SemaphoreType.DMA,
        pltpu.SemaphoreType.DMA,
    ],
)
def _scatter_kernel(y_hbm, src_hbm, dst_hbm, out_hbm,
                    idxs0_v, idxs1_v, idxd0_v, idxd1_v, rows0_v, rows1_v,
                    part_sh, semg0, semg1, semis0, semis1, semid0, semid1):
    cid = lax.axis_index("c")
    sid = lax.axis_index("s")
    wid = sid * _NC + cid

    def zbody(i, _):
        r = i // (_D // 16)
        col = (i % (_D // 16)) * 16
        rows0_v[r, pl.ds(col, 16)] = jnp.zeros((16,), jnp.float32)
        return 0
    lax.fori_loop(0, _C * (_D // 16), zbody, 0)

    def zinit(t, _):
        pltpu.sync_copy(rows0_v, part_sh.at[pl.ds(sid * _RPT + t * _C, _C)])
        return 0
    lax.fori_loop(0, _RPT // _C, zinit, 0)

    pltpu.sync_copy(src_hbm.at[wid, 0], idxs0_v)
    pltpu.sync_copy(dst_hbm.at[wid, 0], idxd0_v)
    pltpu.sync_copy(src_hbm.at[wid, 1], idxs1_v)
    pltpu.sync_copy(dst_hbm.at[wid, 1], idxd1_v)
    pltpu.async_copy(y_hbm.at[idxs0_v], rows0_v, semg0)
    pltpu.async_copy(y_hbm.at[idxs1_v], rows1_v, semg1)
    plsc.subcore_barrier()

    def pair(t, _):
        j = 2 * t
        pltpu.make_async_copy(y_hbm.at[idxs0_v], rows0_v, semg0).wait()

        @pl.when(j + 2 < _NCH)
        def _():
            pltpu.async_copy(src_hbm.at[wid, j + 2], idxs0_v, semis0)
        pltpu.sync_copy(rows0_v, part_sh.at[idxd0_v], add=True)

        @pl.when(j + 2 < _NCH)
        def _():
            pltpu.async_copy(dst_hbm.at[wid, j + 2], idxd0_v, semid0)
            pltpu.make_async_copy(src_hbm.at[wid, j + 2], idxs0_v, semis0).wait()
            pltpu.async_copy(y_hbm.at[idxs0_v], rows0_v, semg0)
        pltpu.make_async_copy(y_hbm.at[idxs1_v], rows1_v, semg1).wait()

        @pl.when(j + 3 < _NCH)
        def _():
            pltpu.async_copy(src_hbm.at[wid, j + 3], idxs1_v, semis1)
        pltpu.sync_copy(rows1_v, part_sh.at[idxd1_v], add=True)

        @pl.when(j + 3 < _NCH)
        def _():
            pltpu.async_copy(dst_hbm.at[wid, j + 3], idxd1_v, semid1)
            pltpu.make_async_copy(src_hbm.at[wid, j + 3], idxs1_v, semis1).wait()
            pltpu.async_copy(y_hbm.at[idxs1_v], rows1_v, semg1)
            pltpu.make_async_copy(dst_hbm.at[wid, j + 3], idxd1_v, semid1).wait()

        @pl.when(j + 2 < _NCH)
        def _():
            pltpu.make_async_copy(dst_hbm.at[wid, j + 2], idxd0_v, semid0).wait()
        return 0
    lax.fori_loop(0, _NCH // 2, pair, 0)
    plsc.subcore_barrier()

    def wout(t, _):
        r0 = sid * _RPT + t * _C
        pltpu.sync_copy(part_sh.at[pl.ds(r0, _C)], out_hbm.at[cid, pl.ds(r0, _C)])
        return 0
    lax.fori_loop(0, _RPT // _C, wout, 0)


_BR = 1024
_NB = _NPAD // _BR
_GR = _BR // 128


def _dinv_col(degp_ref, i):
    g = degp_ref[0] + degp_ref[1] + 1.0
    r = jnp.broadcast_to(g[:, None, :], (_GR, 128, 128)).reshape(_BR, 128)
    lane = lax.broadcasted_iota(jnp.int32, (_BR, 128), 1)
    row = lax.broadcasted_iota(jnp.int32, (_BR, 128), 0) % 128
    deg = jnp.sum(jnp.where(lane == row, r, 0.0), axis=1, keepdims=True)
    return lax.rsqrt(deg)


def _tc1_body(degp_ref, x_ref, w1_ref, y1_ref):
    dinv = _dinv_col(degp_ref, None)
    xw = jnp.dot(x_ref[...], w1_ref[...], preferred_element_type=jnp.float32)
    y1_ref[...] = xw * dinv


def _tc1(degp, x, W1):
    return pl.pallas_call(
        _tc1_body,
        grid=(_NB,),
        in_specs=[
            pl.BlockSpec((2, _GR, 128), lambda i: (0, i, 0)),
            pl.BlockSpec((_BR, _D), lambda i: (i, 0)),
            pl.BlockSpec((_D, _D), lambda i: (0, 0)),
        ],
        out_specs=pl.BlockSpec((_BR, _D), lambda i: (i, 0)),
        out_shape=jax.ShapeDtypeStruct((_N, _D), jnp.float32),
    )(degp, x, W1)


def _tc2_body(degp_ref, parts_ref, y1_ref, b1_ref, w2_ref, y2_ref):
    dinv = _dinv_col(degp_ref, None)
    s = parts_ref[0] + parts_ref[1] + y1_ref[...]
    h = jnp.maximum(dinv * s + b1_ref[...], 0.0)
    y2_ref[...] = dinv * jnp.dot(
        h, w2_ref[...], preferred_element_type=jnp.float32)


def _tc2(degp, parts1, y1, b1, W2):
    return pl.pallas_call(
        _tc2_body,
        grid=(_NB,),
        in_specs=[
            pl.BlockSpec((2, _GR, 128), lambda i: (0, i, 0)),
            pl.BlockSpec((2, _BR, _D), lambda i: (0, i, 0)),
            pl.BlockSpec((_BR, _D), lambda i: (i, 0)),
            pl.BlockSpec((1, _D), lambda i: (0, 0)),
            pl.BlockSpec((_D, _D), lambda i: (0, 0)),
        ],
        out_specs=pl.BlockSpec((_BR, _D), lambda i: (i, 0)),
        out_shape=jax.ShapeDtypeStruct((_N, _D), jnp.float32),
    )(degp, parts1, y1, b1.reshape(1, _D), W2)


def _tc3_body(degp_ref, parts_ref, y2_ref, b2_ref, wk_ref, lt_ref,
              s_ref, logits_ref):
    dinv = _dinv_col(degp_ref, None)
    s = parts_ref[0] + parts_ref[1] + y2_ref[...]
    agg = dinv * s + b2_ref[...]
    logits = jnp.dot(agg, wk_ref[...], preferred_element_type=jnp.float32)
    logits_ref[...] = logits
    tau = jnp.exp(lt_ref[0])
    z = logits / tau
    z = z - jnp.max(z, axis=-1, keepdims=True)
    ez = jnp.exp(z)
    s_ref[...] = ez / jnp.sum(ez, axis=-1, keepdims=True)


def _tc3(degp, parts2, y2, b2, Wk, log_tau):
    return pl.pallas_call(
        _tc3_body,
        grid=(_NB,),
        in_specs=[
            pl.BlockSpec((2, _GR, 128), lambda i: (0, i, 0)),
            pl.BlockSpec((2, _BR, _D), lambda i: (0, i, 0)),
            pl.BlockSpec((_BR, _D), lambda i: (i, 0)),
            pl.BlockSpec((1, _D), lambda i: (0, 0)),
            pl.BlockSpec((_D, _K), lambda i: (0, 0)),
            pl.BlockSpec(memory_space=pltpu.SMEM),
        ],
        out_specs=[
            pl.BlockSpec((_BR, _K), lambda i: (i, 0)),
            pl.BlockSpec((_BR, _K), lambda i: (i, 0)),
        ],
        out_shape=[
            jax.ShapeDtypeStruct((_N, _K), jnp.float32),
            jax.ShapeDtypeStruct((_N, _K), jnp.float32),
        ],
    )(degp, parts2, y2, b2.reshape(1, _D), Wk, log_tau.reshape(1))


def kernel(x, edge_index, W1, b1, W2, b2, Wk, log_tau):
    pad = _EPAD - _E
    src = jnp.concatenate(
        [edge_index[0], jnp.zeros((pad,), jnp.int32)]).reshape(_NW, _NCH, _C)
    dst = jnp.concatenate(
        [edge_index[1], jnp.full((pad,), _N + 200, jnp.int32)]
    ).reshape(_NW, _NCH, _C)

    degp = _deg_kernel(dst).reshape(_NC, _NPAD // 128, 128)
    y1 = _tc1(degp, x, W1)
    parts1 = _scatter_kernel(y1, src, dst)
    y2 = _tc2(degp, parts1, y1, b1, W2)
    parts2 = _scatter_kernel(y2, src, dst)
    S, logits = _tc3(degp, parts2, y2, b2, Wk, log_tau)
    return S, logits


# speedup vs baseline: 2.9486x; 2.9486x over previous
import functools

import jax
import jax.numpy as jnp
from jax import lax
from jax.experimental import pallas as pl
from jax.experimental.pallas import tpu as pltpu
from jax.experimental.pallas import tpu_sc as plsc

_N = 10000
_E = 320000
_D = 128
_K = 64

_NC = 2
_NS = 16
_NW = _NC * _NS
_EPW = _E // _NW
_C = 80
_NCH = _EPW // _C
_NPAD = 10240
_RPT = _NPAD // _NS

_mesh = plsc.VectorSubcoreMesh(core_axis_name="c", subcore_axis_name="s")


@functools.partial(
    pl.kernel,
    out_type=jax.ShapeDtypeStruct((_NC * _NPAD,), jnp.float32),
    mesh=_mesh,
    scratch_types=[
        pltpu.VMEM((_NCH, _C), jnp.int32),
        pltpu.VMEM((_C,), jnp.float32),
        pltpu.VMEM((_RPT,), jnp.float32),
        pltpu.VMEM_SHARED((_NPAD,), jnp.float32),
        pltpu.SemaphoreType.DMA,
    ],
)
def _deg_kernel(dst_hbm, out_hbm, idxd_v, ones_v, zbuf_v, part_sh, sem):
    cid = lax.axis_index("c")
    sid = lax.axis_index("s")
    wid = sid * _NC + cid

    pltpu.async_copy(dst_hbm.at[wid], idxd_v, sem)

    def fo(k, _):
        ones_v[pl.ds(k * 16, 16)] = jnp.ones((16,), jnp.float32)
        return 0
    lax.fori_loop(0, _C // 16, fo, 0)

    def fz(k, _):
        zbuf_v[pl.ds(k * 16, 16)] = jnp.zeros((16,), jnp.float32)
        return 0
    lax.fori_loop(0, _RPT // 16, fz, 0)

    pltpu.sync_copy(zbuf_v, part_sh.at[pl.ds(sid * _RPT, _RPT)])
    pltpu.make_async_copy(dst_hbm.at[wid], idxd_v, sem).wait()
    plsc.subcore_barrier()

    def chunk(t, _):
        j = 5 * t
        for q in range(5):
            pltpu.async_copy(ones_v, part_sh.at[idxd_v.at[j + q]], sem,
                             add=True)
        for q in range(5):
            pltpu.make_async_copy(ones_v, part_sh.at[idxd_v.at[j + q]],
                                  sem).wait()
        return 0
    lax.fori_loop(0, _NCH // 5, chunk, 0)
    plsc.subcore_barrier()

    pltpu.sync_copy(part_sh.at[pl.ds(sid * _RPT, _RPT)],
                    out_hbm.at[pl.ds(cid * _NPAD + sid * _RPT, _RPT)])


@functools.partial(
    pl.kernel,
    out_type=jax.ShapeDtypeStruct((_NC, _NPAD, _D), jnp.float32),
    mesh=_mesh,
    scratch_types=[
        pltpu.VMEM((_NCH, _C), jnp.int32),
        pltpu.VMEM((_C,), jnp.int32),
        pltpu.VMEM((_C,), jnp.int32),
        pltpu.VMEM((_C, _D), jnp.float32),
        pltpu.VMEM((_C, _D), jnp.float32),
        pltpu.VMEM_SHARED((_NPAD, _D), jnp.float32),
        pltpu.SemaphoreType.DMA,
        pltpu.SemaphoreType.DMA,
        pltpu.SemaphoreType.DMA,
        pltpu.SemaphoreType.DMA,
    ],
)
def _scatter_kernel(y_hbm, src_hbm, dst_hbm, out_hbm,
                    idxs_v, idxd0_v, idxd1_v, rows0_v, rows1_v, part_sh,
                    sem0, sem1, semd0, semd1):
    cid = lax.axis_index("c")
    sid = lax.axis_index("s")
    wid = sid * _NC + cid

    pltpu.async_copy(src_hbm.at[wid], idxs_v, semd0)
    pltpu.async_copy(dst_hbm.at[wid, 0], idxd0_v, semd1)

    def zbody(i, _):
        r = i // (_D // 16)
        col = (i % (_D // 16)) * 16
        rows0_v[r, pl.ds(col, 16)] = jnp.zeros((16,), jnp.float32)
        return 0
    lax.fori_loop(0, _C * (_D // 16), zbody, 0)

    for t in range(_RPT // _C):
        pltpu.async_copy(rows0_v, part_sh.at[pl.ds(sid * _RPT + t * _C, _C)],
                         sem1)
    for t in range(_RPT // _C):
        pltpu.make_async_copy(
            rows0_v, part_sh.at[pl.ds(sid * _RPT + t * _C, _C)], sem1).wait()

    pltpu.make_async_copy(src_hbm.at[wid], idxs_v, semd0).wait()
    pltpu.make_async_copy(dst_hbm.at[wid, 0], idxd0_v, semd1).wait()
    pltpu.async_copy(y_hbm.at[idxs_v.at[0]], rows0_v, sem0)
    plsc.subcore_barrier()

    def pair(t, _):
        j = 2 * t
        pltpu.async_copy(y_hbm.at[idxs_v.at[j + 1]], rows1_v, sem1)
        pltpu.async_copy(dst_hbm.at[wid, j + 1], idxd1_v, semd1)
        pltpu.make_async_copy(y_hbm.at[idxs_v.at[j]], rows0_v, sem0).wait()
        pltpu.sync_copy(rows0_v, part_sh.at[idxd0_v], add=True)
        pltpu.async_copy(y_hbm.at[idxs_v.at[j + 2]], rows0_v, sem0)
        pltpu.async_copy(dst_hbm.at[wid, j + 2], idxd0_v, semd0)
        pltpu.make_async_copy(y_hbm.at[idxs_v.at[j + 1]], rows1_v, sem1).wait()
        pltpu.make_async_copy(dst_hbm.at[wid, j + 1], idxd1_v, semd1).wait()
        pltpu.sync_copy(rows1_v, part_sh.at[idxd1_v], add=True)
        pltpu.make_async_copy(dst_hbm.at[wid, j + 2], idxd0_v, semd0).wait()
        return 0
    lax.fori_loop(0, (_NCH - 1) // 2, pair, 0)
    pltpu.make_async_copy(y_hbm.at[idxs_v.at[_NCH - 1]], rows0_v, sem0).wait()
    pltpu.sync_copy(rows0_v, part_sh.at[idxd0_v], add=True)
    plsc.subcore_barrier()

    for t in range(_RPT // _C):
        r0 = sid * _RPT + t * _C
        pltpu.async_copy(part_sh.at[pl.ds(r0, _C)],
                         out_hbm.at[cid, pl.ds(r0, _C)], sem0)
    for t in range(_RPT // _C):
        r0 = sid * _RPT + t * _C
        pltpu.make_async_copy(part_sh.at[pl.ds(r0, _C)],
                              out_hbm.at[cid, pl.ds(r0, _C)], sem0).wait()


_BR = 1024
_NB = _NPAD // _BR
_GR = _BR // 128


def _dinv_col(degp_ref, i):
    g = degp_ref[0] + degp_ref[1] + 1.0
    r = jnp.broadcast_to(g[:, None, :], (_GR, 128, 128)).reshape(_BR, 128)
    lane = lax.broadcasted_iota(jnp.int32, (_BR, 128), 1)
    row = lax.broadcasted_iota(jnp.int32, (_BR, 128), 0) % 128
    deg = jnp.sum(jnp.where(lane == row, r, 0.0), axis=1, keepdims=True)
    return lax.rsqrt(deg)


def _tc1_body(degp_ref, x_ref, w1_ref, y1_ref):
    dinv = _dinv_col(degp_ref, None)
    xw = jnp.dot(x_ref[...], w1_ref[...], preferred_element_type=jnp.float32)
    y1_ref[...] = xw * dinv


def _tc1(degp, x, W1):
    return pl.pallas_call(
        _tc1_body,
        grid=(_NB,),
        in_specs=[
            pl.BlockSpec((2, _GR, 128), lambda i: (0, i, 0)),
            pl.BlockSpec((_BR, _D), lambda i: (i, 0)),
            pl.BlockSpec((_D, _D), lambda i: (0, 0)),
        ],
        out_specs=pl.BlockSpec((_BR, _D), lambda i: (i, 0)),
        out_shape=jax.ShapeDtypeStruct((_N, _D), jnp.float32),
    )(degp, x, W1)


def _tc2_body(degp_ref, parts_ref, y1_ref, b1_ref, w2_ref, y2_ref):
    dinv = _dinv_col(degp_ref, None)
    s = parts_ref[0] + parts_ref[1] + y1_ref[...]
    h = jnp.maximum(dinv * s + b1_ref[...], 0.0)
    y2_ref[...] = dinv * jnp.dot(
        h, w2_ref[...], preferred_element_type=jnp.float32)


def _tc2(degp, parts1, y1, b1, W2):
    return pl.pallas_call(
        _tc2_body,
        grid=(_NB,),
        in_specs=[
            pl.BlockSpec((2, _GR, 128), lambda i: (0, i, 0)),
            pl.BlockSpec((2, _BR, _D), lambda i: (0, i, 0)),
            pl.BlockSpec((_BR, _D), lambda i: (i, 0)),
            pl.BlockSpec((1, _D), lambda i: (0, 0)),
            pl.BlockSpec((_D, _D), lambda i: (0, 0)),
        ],
        out_specs=pl.BlockSpec((_BR, _D), lambda i: (i, 0)),
        out_shape=jax.ShapeDtypeStruct((_N, _D), jnp.float32),
    )(degp, parts1, y1, b1.reshape(1, _D), W2)


def _tc3_body(degp_ref, parts_ref, y2_ref, b2_ref, wk_ref, lt_ref,
              s_ref, logits_ref):
    dinv = _dinv_col(degp_ref, None)
    s = parts_ref[0] + parts_ref[1] + y2_ref[...]
    agg = dinv * s + b2_ref[...]
    logits = jnp.dot(agg, wk_ref[...], preferred_element_type=jnp.float32)
    logits_ref[...] = logits
    tau = jnp.exp(lt_ref[0])
    z = logits / tau
    z = z - jnp.max(z, axis=-1, keepdims=True)
    ez = jnp.exp(z)
    s_ref[...] = ez / jnp.sum(ez, axis=-1, keepdims=True)


def _tc3(degp, parts2, y2, b2, Wk, log_tau):
    return pl.pallas_call(
        _tc3_body,
        grid=(_NB,),
        in_specs=[
            pl.BlockSpec((2, _GR, 128), lambda i: (0, i, 0)),
            pl.BlockSpec((2, _BR, _D), lambda i: (0, i, 0)),
            pl.BlockSpec((_BR, _D), lambda i: (i, 0)),
            pl.BlockSpec((1, _D), lambda i: (0, 0)),
            pl.BlockSpec((_D, _K), lambda i: (0, 0)),
            pl.BlockSpec(memory_space=pltpu.SMEM),
        ],
        out_specs=[
            pl.BlockSpec((_BR, _K), lambda i: (i, 0)),
            pl.BlockSpec((_BR, _K), lambda i: (i, 0)),
        ],
        out_shape=[
            jax.ShapeDtypeStruct((_N, _K), jnp.float32),
            jax.ShapeDtypeStruct((_N, _K), jnp.float32),
        ],
    )(degp, parts2, y2, b2.reshape(1, _D), Wk, log_tau.reshape(1))


def kernel(x, edge_index, W1, b1, W2, b2, Wk, log_tau):
    src = edge_index[0].reshape(_NW, _NCH, _C)
    dst = edge_index[1].reshape(_NW, _NCH, _C)

    degp = _deg_kernel(dst).reshape(_NC, _NPAD // 128, 128)
    y1 = _tc1(degp, x, W1)
    parts1 = _scatter_kernel(y1, src, dst)
    y2 = _tc2(degp, parts1, y1, b1, W2)
    parts2 = _scatter_kernel(y2, src, dst)
    S, logits = _tc3(degp, parts2, y2, b2, Wk, log_tau)
    return S, logits


# speedup vs baseline: 2.9542x; 1.0019x over previous
import functools

import jax
import jax.numpy as jnp
from jax import lax
from jax.experimental import pallas as pl
from jax.experimental.pallas import tpu as pltpu
from jax.experimental.pallas import tpu_sc as plsc

_N = 10000
_E = 320000
_D = 128
_K = 64

_NC = 2
_NS = 16
_NW = _NC * _NS
_EPW = _E // _NW
_C = 80
_NCH = _EPW // _C
_NPAD = 10240
_RPT = _NPAD // _NS

_mesh = plsc.VectorSubcoreMesh(core_axis_name="c", subcore_axis_name="s")


@functools.partial(
    pl.kernel,
    out_type=jax.ShapeDtypeStruct((_NC * _NPAD,), jnp.float32),
    mesh=_mesh,
    scratch_types=[
        pltpu.VMEM((_NCH, _C), jnp.int32),
        pltpu.VMEM((_C,), jnp.float32),
        pltpu.VMEM((_RPT,), jnp.float32),
        pltpu.VMEM_SHARED((_NPAD,), jnp.float32),
        pltpu.SemaphoreType.DMA,
    ],
)
def _deg_kernel(dst_hbm, out_hbm, idxd_v, ones_v, zbuf_v, part_sh, sem):
    cid = lax.axis_index("c")
    sid = lax.axis_index("s")
    wid = sid * _NC + cid

    pltpu.async_copy(dst_hbm.at[wid], idxd_v, sem)

    def fo(k, _):
        ones_v[pl.ds(k * 16, 16)] = jnp.ones((16,), jnp.float32)
        return 0
    lax.fori_loop(0, _C // 16, fo, 0)

    def fz(k, _):
        zbuf_v[pl.ds(k * 16, 16)] = jnp.zeros((16,), jnp.float32)
        return 0
    lax.fori_loop(0, _RPT // 16, fz, 0)

    pltpu.sync_copy(zbuf_v, part_sh.at[pl.ds(sid * _RPT, _RPT)])
    pltpu.make_async_copy(dst_hbm.at[wid], idxd_v, sem).wait()
    plsc.subcore_barrier()

    def chunk(t, _):
        j = 25 * t
        for q in range(25):
            pltpu.async_copy(ones_v, part_sh.at[idxd_v.at[j + q]], sem,
                             add=True)
        for q in range(25):
            pltpu.make_async_copy(ones_v, part_sh.at[idxd_v.at[j + q]],
                                  sem).wait()
        return 0
    lax.fori_loop(0, _NCH // 25, chunk, 0)
    plsc.subcore_barrier()

    pltpu.sync_copy(part_sh.at[pl.ds(sid * _RPT, _RPT)],
                    out_hbm.at[pl.ds(cid * _NPAD + sid * _RPT, _RPT)])


@functools.partial(
    pl.kernel,
    out_type=jax.ShapeDtypeStruct((_NC, _NPAD, _D), jnp.float32),
    mesh=_mesh,
    scratch_types=[
        pltpu.VMEM((_NCH, _C), jnp.int32),
        pltpu.VMEM((_C,), jnp.int32),
        pltpu.VMEM((_C,), jnp.int32),
        pltpu.VMEM((_C, _D), jnp.float32),
        pltpu.VMEM((_C, _D), jnp.float32),
        pltpu.VMEM_SHARED((_NPAD, _D), jnp.float32),
        pltpu.SemaphoreType.DMA,
        pltpu.SemaphoreType.DMA,
        pltpu.SemaphoreType.DMA,
        pltpu.SemaphoreType.DMA,
    ],
)
def _scatter_kernel(y_hbm, src_hbm, dst_hbm, out_hbm,
                    idxs_v, idxd0_v, idxd1_v, rows0_v, rows1_v, part_sh,
                    sem0, sem1, semd0, semd1):
    cid = lax.axis_index("c")
    sid = lax.axis_index("s")
    wid = sid * _NC + cid

    pltpu.async_copy(src_hbm.at[wid], idxs_v, semd0)
    pltpu.async_copy(dst_hbm.at[wid, 0], idxd0_v, semd1)

    def zbody(i, _):
        r = i // (_D // 16)
        col = (i % (_D // 16)) * 16
        rows0_v[r, pl.ds(col, 16)] = jnp.zeros((16,), jnp.float32)
        return 0
    lax.fori_loop(0, _C * (_D // 16), zbody, 0)

    for t in range(_RPT // _C):
        pltpu.async_copy(rows0_v, part_sh.at[pl.ds(sid * _RPT + t * _C, _C)],
                         sem1)
    for t in range(_RPT // _C):
        pltpu.make_async_copy(
            rows0_v, part_sh.at[pl.ds(sid * _RPT + t * _C, _C)], sem1).wait()

    pltpu.make_async_copy(src_hbm.at[wid], idxs_v, semd0).wait()
    pltpu.make_async_copy(dst_hbm.at[wid, 0], idxd0_v, semd1).wait()
    pltpu.async_copy(y_hbm.at[idxs_v.at[0]], rows0_v, sem0)
    plsc.subcore_barrier()

    def pair(t, _):
        j = 2 * t
        pltpu.async_copy(y_hbm.at[idxs_v.at[j + 1]], rows1_v, sem1)
        pltpu.async_copy(dst_hbm.at[wid, j + 1], idxd1_v, semd1)
        pltpu.make_async_copy(y_hbm.at[idxs_v.at[j]], rows0_v, sem0).wait()
        pltpu.sync_copy(rows0_v, part_sh.at[idxd0_v], add=True)
        pltpu.async_copy(y_hbm.at[idxs_v.at[j + 2]], rows0_v, sem0)
        pltpu.async_copy(dst_hbm.at[wid, j + 2], idxd0_v, semd0)
        pltpu.make_async_copy(y_hbm.at[idxs_v.at[j + 1]], rows1_v, sem1).wait()
        pltpu.make_async_copy(dst_hbm.at[wid, j + 1], idxd1_v, semd1).wait()
        pltpu.sync_copy(rows1_v, part_sh.at[idxd1_v], add=True)
        pltpu.make_async_copy(dst_hbm.at[wid, j + 2], idxd0_v, semd0).wait()
        return 0
    lax.fori_loop(0, (_NCH - 1) // 2, pair, 0)
    pltpu.make_async_copy(y_hbm.at[idxs_v.at[_NCH - 1]], rows0_v, sem0).wait()
    pltpu.sync_copy(rows0_v, part_sh.at[idxd0_v], add=True)
    plsc.subcore_barrier()

    for t in range(_RPT // _C):
        r0 = sid * _RPT + t * _C
        pltpu.async_copy(part_sh.at[pl.ds(r0, _C)],
                         out_hbm.at[cid, pl.ds(r0, _C)], sem0)
    for t in range(_RPT // _C):
        r0 = sid * _RPT + t * _C
        pltpu.make_async_copy(part_sh.at[pl.ds(r0, _C)],
                              out_hbm.at[cid, pl.ds(r0, _C)], sem0).wait()


_BR = 1024
_NB = _NPAD // _BR
_GR = _BR // 128


def _dinv_col(degp_ref, i):
    g = degp_ref[0] + degp_ref[1] + 1.0
    r = jnp.broadcast_to(g[:, None, :], (_GR, 128, 128)).reshape(_BR, 128)
    lane = lax.broadcasted_iota(jnp.int32, (_BR, 128), 1)
    row = lax.broadcasted_iota(jnp.int32, (_BR, 128), 0) % 128
    deg = jnp.sum(jnp.where(lane == row, r, 0.0), axis=1, keepdims=True)
    return lax.rsqrt(deg)


def _tc1_body(degp_ref, x_ref, w1_ref, y1_ref):
    dinv = _dinv_col(degp_ref, None)
    xw = jnp.dot(x_ref[...], w1_ref[...], preferred_element_type=jnp.float32)
    y1_ref[...] = xw * dinv


def _tc1(degp, x, W1):
    return pl.pallas_call(
        _tc1_body,
        grid=(_NB,),
        in_specs=[
            pl.BlockSpec((2, _GR, 128), lambda i: (0, i, 0)),
            pl.BlockSpec((_BR, _D), lambda i: (i, 0)),
            pl.BlockSpec((_D, _D), lambda i: (0, 0)),
        ],
        out_specs=pl.BlockSpec((_BR, _D), lambda i: (i, 0)),
        out_shape=jax.ShapeDtypeStruct((_N, _D), jnp.float32),
    )(degp, x, W1)


def _tc2_body(degp_ref, parts_ref, y1_ref, b1_ref, w2_ref, y2_ref):
    dinv = _dinv_col(degp_ref, None)
    s = parts_ref[0] + parts_ref[1] + y1_ref[...]
    h = jnp.maximum(dinv * s + b1_ref[...], 0.0)
    y2_ref[...] = dinv * jnp.dot(
        h, w2_ref[...], preferred_element_type=jnp.float32)


def _tc2(degp, parts1, y1, b1, W2):
    return pl.pallas_call(
        _tc2_body,
        grid=(_NB,),
        in_specs=[
            pl.BlockSpec((2, _GR, 128), lambda i: (0, i, 0)),
            pl.BlockSpec((2, _BR, _D), lambda i: (0, i, 0)),
            pl.BlockSpec((_BR, _D), lambda i: (i, 0)),
            pl.BlockSpec((1, _D), lambda i: (0, 0)),
            pl.BlockSpec((_D, _D), lambda i: (0, 0)),
        ],
        out_specs=pl.BlockSpec((_BR, _D), lambda i: (i, 0)),
        out_shape=jax.ShapeDtypeStruct((_N, _D), jnp.float32),
    )(degp, parts1, y1, b1.reshape(1, _D), W2)


def _tc3_body(degp_ref, parts_ref, y2_ref, b2_ref, wk_ref, lt_ref,
              s_ref, logits_ref):
    dinv = _dinv_col(degp_ref, None)
    s = parts_ref[0] + parts_ref[1] + y2_ref[...]
    agg = dinv * s + b2_ref[...]
    logits = jnp.dot(agg, wk_ref[...], preferred_element_type=jnp.float32)
    logits_ref[...] = logits
    tau = jnp.exp(lt_ref[0])
    z = logits / tau
    z = z - jnp.max(z, axis=-1, keepdims=True)
    ez = jnp.exp(z)
    s_ref[...] = ez / jnp.sum(ez, axis=-1, keepdims=True)


def _tc3(degp, parts2, y2, b2, Wk, log_tau):
    return pl.pallas_call(
        _tc3_body,
        grid=(_NB,),
        in_specs=[
            pl.BlockSpec((2, _GR, 128), lambda i: (0, i, 0)),
            pl.BlockSpec((2, _BR, _D), lambda i: (0, i, 0)),
            pl.BlockSpec((_BR, _D), lambda i: (i, 0)),
            pl.BlockSpec((1, _D), lambda i: (0, 0)),
            pl.BlockSpec((_D, _K), lambda i: (0, 0)),
            pl.BlockSpec(memory_space=pltpu.SMEM),
        ],
        out_specs=[
            pl.BlockSpec((_BR, _K), lambda i: (i, 0)),
            pl.BlockSpec((_BR, _K), lambda i: (i, 0)),
        ],
        out_shape=[
            jax.ShapeDtypeStruct((_N, _K), jnp.float32),
            jax.ShapeDtypeStruct((_N, _K), jnp.float32),
        ],
    )(degp, parts2, y2, b2.reshape(1, _D), Wk, log_tau.reshape(1))


def kernel(x, edge_index, W1, b1, W2, b2, Wk, log_tau):
    src = edge_index[0].reshape(_NW, _NCH, _C)
    dst = edge_index[1].reshape(_NW, _NCH, _C)

    degp = _deg_kernel(dst).reshape(_NC, _NPAD // 128, 128)
    y1 = _tc1(degp, x, W1)
    parts1 = _scatter_kernel(y1, src, dst)
    y2 = _tc2(degp, parts1, y1, b1, W2)
    parts2 = _scatter_kernel(y2, src, dst)
    S, logits = _tc3(degp, parts2, y2, b2, Wk, log_tau)
    return S, logits
